# baseline (speedup 1.0000x reference)
import jax
import jax.numpy as jnp
from jax import lax
from jax.experimental import pallas as pl
from jax.experimental.pallas import tpu as pltpu
from jax.experimental.pallas import tpu_sc as plsc

ROWS, N = 64, 8192
K = int(0.05 * N)
NC, NS, L = 2, 16, 16
NW = NC * NS
ROWS_PER_W = ROWS // NW
NVEC = N // L
INT_MIN = -2147483648


def _ordered_bits_to_f32(cand_u):
    bits = jnp.where(cand_u < 0, cand_u ^ INT_MIN, ~cand_u)
    return lax.bitcast_convert_type(bits, jnp.float32)


def _body(x_hbm, out_hbm, x_v, cand_a, cand_b, sem):
    wid = lax.axis_index("s") * NC + lax.axis_index("c")
    base = wid * ROWS_PER_W
    pltpu.sync_copy(x_hbm.at[pl.ds(base, ROWS_PER_W)], x_v)

    UNROLL = 8
    S1 = 9
    S2 = 8
    S3 = 32 - S1 - S2
    ONE = jnp.full((L,), 1, jnp.int32)
    ZERO = jnp.full((L,), 0, jnp.int32)
    LANE = lax.iota(jnp.int32, L)

    def lane_sum(acc):
        cnt = acc[0]
        for e in range(1, L):
            cnt = cnt + acc[e]
        return cnt

    def popcnt(m):
        return plsc.all_reduce_population_count(m)[0]

    for r in range(ROWS_PER_W):
        def bit_step(b, carry):
            prefix_u, cnt_p = carry
            cand_u = prefix_u | lax.shift_left(jnp.int32(1), 31 - b)
            cand_f = _ordered_bits_to_f32(cand_u)

            def count(j, accs):
                new = []
                for u in range(UNROLL):
                    xv = x_v[r, pl.ds((j * UNROLL + u) * L, L)]
                    new.append(accs[u] +
                               jnp.where(xv >= cand_f, ONE, ZERO))
                return tuple(new)

            accs = lax.fori_loop(0, NVEC // UNROLL, count,
                                 tuple(ZERO for _ in range(UNROLL)))
            acc = accs[0]
            for u in range(1, UNROLL):
                acc = acc + accs[u]
            cnt = lane_sum(acc)
            keep = cnt >= K
            return (jnp.where(keep, cand_u, prefix_u),
                    jnp.where(keep, cnt, cnt_p))

        prefix_u, cnt_p = lax.fori_loop(
            0, S1, bit_step, (jnp.int32(0), jnp.int32(0)))

        f_lo = _ordered_bits_to_f32(prefix_u)
        f_hi = _ordered_bits_to_f32(prefix_u + jnp.int32(1 << (32 - S1)))

        def compact1(j, off):
            for u in range(4):
                xv = x_v[r, pl.ds((j * 4 + u) * L, L)]
                m = (xv >= f_lo) & jnp.logical_not(xv >= f_hi)
                plsc.store_compressed(cand_a.at[pl.ds(off, L)], xv, mask=m)
                off = off + popcnt(m)
            return off

        n_w = lax.fori_loop(0, NVEC // 4, compact1, jnp.int32(0))
        above = cnt_p - n_w

        def make_cand_steps(buf, n_cand, n_above, start_bit):
            tf = n_cand // L
            rem = n_cand - tf * L

            def cstep(b, carry):
                prefix_u, cnt_p = carry
                cand_u = prefix_u | lax.shift_left(jnp.int32(1),
                                                   start_bit - b)
                cand_f = _ordered_bits_to_f32(cand_u)

                def count(j, acc):
                    xv = buf[pl.ds(j * L, L)]
                    return acc + jnp.where(xv >= cand_f, ONE, ZERO)

                acc = lax.fori_loop(0, tf, count, ZERO)
                tailv = buf[pl.ds(tf * L, L)]
                mt = (tailv >= cand_f) & (LANE < rem)
                cnt = n_above + lane_sum(acc) + popcnt(mt)
                keep = cnt >= K
                return (jnp.where(keep, cand_u, prefix_u),
                        jnp.where(keep, cnt, cnt_p))

            return cstep, tf, rem

        cstep2, tf1, rem1 = make_cand_steps(cand_a, n_w, above, 22)
        prefix_u, cnt_p = lax.fori_loop(0, S2, cstep2,
                                        (prefix_u, cnt_p))

        f_lo = _ordered_bits_to_f32(prefix_u)
        f_hi = _ordered_bits_to_f32(prefix_u +
                                    jnp.int32(1 << (32 - S1 - S2)))

        def compact2(j, off):
            xv = cand_a[pl.ds(j * L, L)]
            m = (xv >= f_lo) & jnp.logical_not(xv >= f_hi)
            plsc.store_compressed(cand_b.at[pl.ds(off, L)], xv, mask=m)
            return off + popcnt(m)

        off = lax.fori_loop(0, tf1, compact2, jnp.int32(0))
        tailv = cand_a[pl.ds(tf1 * L, L)]
        mt = ((tailv >= f_lo) & jnp.logical_not(tailv >= f_hi) &
              (LANE < rem1))
        plsc.store_compressed(cand_b.at[pl.ds(off, L)], tailv, mask=mt)
        n_w2 = off + popcnt(mt)
        above2 = cnt_p - n_w2

        cstep3, _, _ = make_cand_steps(cand_b, n_w2, above2, S3 - 1)
        prefix_u, cnt_p = lax.fori_loop(0, S3, cstep3,
                                        (prefix_u, cnt_p))

        thr_f = _ordered_bits_to_f32(prefix_u)

        def mask_pass(j, carry):
            for u in range(UNROLL):
                sl = pl.ds((j * UNROLL + u) * L, L)
                xv = x_v[r, sl]
                x_v[r, sl] = jnp.where(xv >= thr_f, xv, jnp.float32(0.0))
            return carry

        lax.fori_loop(0, NVEC // UNROLL, mask_pass, jnp.int32(0))

    pltpu.sync_copy(x_v, out_hbm.at[pl.ds(base, ROWS_PER_W)])


@jax.jit
def kernel(x):
    mesh = plsc.VectorSubcoreMesh(
        core_axis_name="c", subcore_axis_name="s",
        num_cores=NC, num_subcores=NS)
    f = pl.kernel(
        _body,
        out_type=jax.ShapeDtypeStruct((ROWS, N), jnp.float32),
        mesh=mesh,
        compiler_params=pltpu.CompilerParams(needs_layout_passes=False),
        scratch_types=[
            pltpu.VMEM((ROWS_PER_W, N), jnp.float32),
            pltpu.VMEM((N + L,), jnp.float32),
            pltpu.VMEM((N + L,), jnp.float32),
            pltpu.SemaphoreType.DMA,
        ],
    )
    return f(x)
